# Initial kernel scaffold; baseline (speedup 1.0000x reference)
import jax
import jax.numpy as jnp
from jax.experimental import pallas as pl


def kernel(nodes_1, nodes_2, edge_index_1, edge_index_2, coords_1, coords_2, distance_matrices, monopoles_1, dipoles_1, quadrupoles_1, monopoles_2, dipoles_2, quadrupoles_2, batch_size, params):
    raise NotImplementedError("write your pallas kernel here")


# speedup vs baseline: 18.2193x; 18.2193x over previous
import functools

import jax
import jax.numpy as jnp
import numpy as np
from jax.experimental import pallas as pl

_WIDTHS = np.logspace(-1, 0, 5).astype(np.float32)
_CUTOFF = 10.0


def _swish(x):
    return x * (1.0 / (1.0 + jnp.exp(-x)))


def _softplus(x):
    return jnp.maximum(x, 0.0) + jnp.log1p(jnp.exp(-jnp.abs(x)))


def _rowsum(x):
    return jnp.sum(x, axis=-1, keepdims=True)


def _topo(nodes, snd, rcv, We, be, Wu, bu):
    E = snd.shape[0]
    n = nodes.shape[0]
    h = _swish(jnp.dot(nodes, We, preferred_element_type=jnp.float32) + be)
    col = jax.lax.broadcasted_iota(jnp.int32, (E, n), 1)
    oh_s = (col == snd).astype(jnp.float32)
    oh_r = (col == rcv).astype(jnp.float32)
    hs = jnp.dot(oh_s, h, preferred_element_type=jnp.float32)
    agg = jax.lax.dot_general(oh_r, hs, (((0,), (0,)), ((), ())),
                              preferred_element_type=jnp.float32)
    hc = jnp.concatenate([h, agg], axis=1)
    return _swish(jnp.dot(hc, Wu, preferred_element_type=jnp.float32) + bu)


def _expand_i_rows(mat, ti, offset):
    rows = ti * 96
    r = jax.lax.broadcasted_iota(jnp.int32, (rows, mat.shape[0]), 0)
    c = jax.lax.broadcasted_iota(jnp.int32, (rows, mat.shape[0]), 1)
    ei = (offset + r // 96 == c).astype(jnp.float32)
    return jnp.dot(ei, mat, preferred_element_type=jnp.float32)


def _tile_j(mat, reps):
    return jnp.concatenate([mat] * reps, axis=0)


def _stageA_kernel(nodes_1, nodes_2, e1s, e1r, e2s, e2r, We, be, Wu, bu,
                   Wf1, bf1, Wf2, bf2, Ws1_mid, Wk1_top, ns_out, nk_out,
                   *, ti):
    t = pl.program_id(0)
    h1 = _topo(nodes_1[...], e1s[...], e1r[...], We[...], be[...], Wu[...], bu[...])
    h2 = _topo(nodes_2[...], e2s[...], e2r[...], We[...], be[...], Wu[...], bu[...])
    node_size = h1.shape[1]
    Wf1_top = Wf1[:node_size, :]
    Wf1_bot = Wf1[node_size:, :]
    A12 = jnp.dot(h1, Wf1_top, preferred_element_type=jnp.float32)
    A21 = jnp.dot(h1, Wf1_bot, preferred_element_type=jnp.float32)
    B12 = jnp.dot(h2, Wf1_bot, preferred_element_type=jnp.float32)
    B21 = jnp.dot(h2, Wf1_top, preferred_element_type=jnp.float32)
    pre12 = _expand_i_rows(A12, ti, t * ti) + _tile_j(B12, ti) + bf1[...]
    pre21 = _expand_i_rows(A21, ti, t * ti) + _tile_j(B21, ti) + bf1[...]
    u12 = _swish(jnp.dot(_swish(pre12), Wf2[...],
                         preferred_element_type=jnp.float32) + bf2[...])
    u21 = _swish(jnp.dot(_swish(pre21), Wf2[...],
                         preferred_element_type=jnp.float32) + bf2[...])
    nf = u12 + u21
    ns_out[...] = jnp.dot(nf, Ws1_mid[...], preferred_element_type=jnp.float32)
    nk_out[...] = jnp.dot(nf, Wk1_top[...], preferred_element_type=jnp.float32)


def _stageB_kernel(ns, nk, dist, c1, c2, m1, d1, q1, m2, d2, q2, widths,
                   Ws1_G, Ws1_D, Wk1_D, bs1, bk1, Ws2, bs2, Wk2, bk2,
                   Ws3, bs3, Wk3, bk3, out, *, ti):
    t = pl.program_id(1)
    R = dist[0]
    R2 = R * R
    c1r = _expand_i_rows(c1[0], ti, 0)
    m1r = _expand_i_rows(m1[0], ti, 0)
    d1r = _expand_i_rows(d1[0], ti, 0)
    q1r = _expand_i_rows(q1[0], ti, 0)
    c2r = _tile_j(c2[0], ti)
    m2r = _tile_j(m2[0], ti)
    d2r = _tile_j(d2[0], ti)
    q2r = _tile_j(q2[0], ti)
    Rx = (c2r - c1r) / R
    D1 = _rowsum(d1r * Rx)
    D2 = _rowsum(d2r * Rx)
    dd = _rowsum(d1r * d2r)
    qq = _rowsum(q1r * q2r)
    Q1v = jnp.concatenate(
        [_rowsum(q1r[:, 3 * k:3 * k + 3] * Rx) for k in range(3)], axis=1)
    Q2v = jnp.concatenate(
        [_rowsum(q2r[:, 3 * k:3 * k + 3] * Rx) for k in range(3)], axis=1)
    Q1R2 = _rowsum(Q1v * Rx)
    Q2R2 = _rowsum(Q2v * Rx)
    Q1d2 = _rowsum(Q1v * d2r)
    Q2d1 = _rowsum(Q2v * d1r)
    QQv = _rowsum(Q1v * Q2v)
    G = jnp.concatenate([
        m1r * m2r,
        D1 * m2r - D2 * m1r,
        dd,
        -(D1 * D2),
        2.0 * Q1d2 - 2.0 * Q2d1,
        Q1R2 * m2r + Q2R2 * m1r,
        2.0 * qq,
        -4.0 * QQv,
        -Q1R2 * D2 + Q2R2 * D1,
        Q1R2 * Q2R2,
    ], axis=1)
    tt = jnp.clip(R - (_CUTOFF - 1.0), 0.0, 1.0)
    sw = 1.0 - tt * tt * tt * (tt * (tt * 6.0 - 15.0) + 10.0)
    distf = jnp.exp(-R2 * widths[...]) * sw
    s1 = _swish(ns[...]
                + jnp.dot(G, Ws1_G[...], preferred_element_type=jnp.float32)
                + jnp.dot(distf, Ws1_D[...], preferred_element_type=jnp.float32)
                + bs1[...])
    s2 = _swish(jnp.dot(s1, Ws2[...], preferred_element_type=jnp.float32) + bs2[...])
    k1 = _swish(nk[...]
                + jnp.dot(distf, Wk1_D[...], preferred_element_type=jnp.float32)
                + bk1[...])
    k2 = _swish(jnp.dot(k1, Wk2[...], preferred_element_type=jnp.float32) + bk2[...])
    s_out = _softplus(jnp.dot(s2, Ws3[...], preferred_element_type=jnp.float32)
                      + bs3[...]) * sw
    k_out = _softplus(jnp.dot(k2, Wk3[...], preferred_element_type=jnp.float32)
                      + bk3[...])
    S2s = s_out[:, 0:1]
    S2a = s_out[:, 1:2]
    K1s = k_out[:, 0:1]
    K2s = k_out[:, 1:2]
    Ka = k_out[:, 2:3]
    V = K1s * S2s / R + K2s * S2s / R2 - Ka * S2a
    e = jnp.sum(V, axis=(0, 1), keepdims=True)[None]

    @pl.when(t == 0)
    def _init():
        out[...] = e

    @pl.when(t != 0)
    def _acc():
        out[...] = out[...] + e


def kernel(nodes_1, nodes_2, edge_index_1, edge_index_2, coords_1, coords_2,
           distance_matrices, monopoles_1, dipoles_1, quadrupoles_1,
           monopoles_2, dipoles_2, quadrupoles_2, batch_size, params):
    B, n1, n2 = distance_matrices.shape
    TI = 16
    T = n1 // TI
    PB = TI * n2
    f32 = jnp.float32

    p = params
    row = lambda v: v.reshape(1, -1).astype(f32)
    e1s = edge_index_1[0].reshape(-1, 1)
    e1r = edge_index_1[1].reshape(-1, 1)
    e2s = edge_index_2[0].reshape(-1, 1)
    e2r = edge_index_2[1].reshape(-1, 1)

    full = lambda a: pl.BlockSpec(a.shape, lambda *_: (0,) * a.ndim)

    Ws1 = p['Ws1']
    Wk1 = p['Wk1']
    Ws1_G = Ws1[:10, :]
    Ws1_mid = Ws1[10:138, :]
    Ws1_D = Ws1[138:143, :]
    Wk1_top = Wk1[:128, :]
    Wk1_D = Wk1[128:133, :]

    stageA_inputs = (nodes_1, nodes_2, e1s, e1r, e2s, e2r,
                     p['We'], row(p['be']), p['Wu'], row(p['bu']),
                     p['Wf1'], row(p['bf1']), p['Wf2'], row(p['bf2']),
                     Ws1_mid, Wk1_top)
    ns_all, nk_all = pl.pallas_call(
        functools.partial(_stageA_kernel, ti=TI),
        grid=(T,),
        in_specs=[full(a) for a in stageA_inputs],
        out_specs=[
            pl.BlockSpec((PB, 128), lambda t: (t, 0)),
            pl.BlockSpec((PB, 128), lambda t: (t, 0)),
        ],
        out_shape=[
            jax.ShapeDtypeStruct((n1 * n2, 128), f32),
            jax.ShapeDtypeStruct((n1 * n2, 128), f32),
        ],
    )(*stageA_inputs)

    dist3 = distance_matrices.reshape(B, n1 * n2, 1)
    q1f = quadrupoles_1.reshape(B, n1, 9)
    q2f = quadrupoles_2.reshape(B, n2, 9)

    ib = lambda C: pl.BlockSpec((1, TI, C), lambda b, t: (b, t, 0))
    jb = lambda C: pl.BlockSpec((1, n2, C), lambda b, t: (b, 0, 0))
    stageB_data = (ns_all, nk_all, dist3, coords_1, coords_2,
                   monopoles_1, dipoles_1, q1f, monopoles_2, dipoles_2, q2f)
    stageB_w = (jnp.asarray(_WIDTHS).reshape(1, 5),
                Ws1_G, Ws1_D, Wk1_D, row(p['bs1']), row(p['bk1']),
                p['Ws2'], row(p['bs2']), p['Wk2'], row(p['bk2']),
                p['Ws3'], row(p['bs3']), p['Wk3'], row(p['bk3']))
    energies = pl.pallas_call(
        functools.partial(_stageB_kernel, ti=TI),
        grid=(B, T),
        in_specs=[
            pl.BlockSpec((PB, 128), lambda b, t: (t, 0)),
            pl.BlockSpec((PB, 128), lambda b, t: (t, 0)),
            pl.BlockSpec((1, PB, 1), lambda b, t: (b, t, 0)),
            ib(3), jb(3), ib(1), ib(3), ib(9), jb(1), jb(3), jb(9),
        ] + [full(a) for a in stageB_w],
        out_specs=pl.BlockSpec((1, 1, 1), lambda b, t: (b, 0, 0)),
        out_shape=jax.ShapeDtypeStruct((B, 1, 1), f32),
    )(*stageB_data, *stageB_w)

    return energies.reshape(B)


# speedup vs baseline: 36.2821x; 1.9914x over previous
import functools

import jax
import jax.numpy as jnp
import numpy as np
from jax.experimental import pallas as pl

_WIDTHS = np.logspace(-1, 0, 5).astype(np.float32)
_CUTOFF = 10.0


def _swish(x):
    return x * (1.0 / (1.0 + jnp.exp(-x)))


def _softplus(x):
    return jnp.maximum(x, 0.0) + jnp.log1p(jnp.exp(-jnp.abs(x)))


def _rowsum(x):
    return jnp.sum(x, axis=-1, keepdims=True)


def _topo(nodes, snd, rcv, We, be, Wu, bu):
    E = snd.shape[0]
    n = nodes.shape[0]
    h = _swish(jnp.dot(nodes, We, preferred_element_type=jnp.float32) + be)
    col = jax.lax.broadcasted_iota(jnp.int32, (E, n), 1)
    oh_s = (col == snd).astype(jnp.float32)
    oh_r = (col == rcv).astype(jnp.float32)
    hs = jnp.dot(oh_s, h, preferred_element_type=jnp.float32)
    agg = jax.lax.dot_general(oh_r, hs, (((0,), (0,)), ((), ())),
                              preferred_element_type=jnp.float32)
    hc = jnp.concatenate([h, agg], axis=1)
    return _swish(jnp.dot(hc, Wu, preferred_element_type=jnp.float32) + bu)


def _expand_i_rows(mat, ti, offset):
    rows = ti * 96
    r = jax.lax.broadcasted_iota(jnp.int32, (rows, mat.shape[0]), 0)
    c = jax.lax.broadcasted_iota(jnp.int32, (rows, mat.shape[0]), 1)
    ei = (offset + r // 96 == c).astype(jnp.float32)
    return jnp.dot(ei, mat, preferred_element_type=jnp.float32)


def _tile_j(mat, reps):
    return jnp.concatenate([mat] * reps, axis=0)


def _stageA_kernel(nodes_1, nodes_2, e1s, e1r, e2s, e2r, We, be, Wu, bu,
                   Wf1, bf1, Wf2, bf2, Ws1_mid, Wk1_top, ns_out, nk_out,
                   *, ti):
    t = pl.program_id(0)
    h1 = _topo(nodes_1[...], e1s[...], e1r[...], We[...], be[...], Wu[...], bu[...])
    h2 = _topo(nodes_2[...], e2s[...], e2r[...], We[...], be[...], Wu[...], bu[...])
    node_size = h1.shape[1]
    Wf1_top = Wf1[:node_size, :]
    Wf1_bot = Wf1[node_size:, :]
    A12 = jnp.dot(h1, Wf1_top, preferred_element_type=jnp.float32)
    A21 = jnp.dot(h1, Wf1_bot, preferred_element_type=jnp.float32)
    B12 = jnp.dot(h2, Wf1_bot, preferred_element_type=jnp.float32)
    B21 = jnp.dot(h2, Wf1_top, preferred_element_type=jnp.float32)
    pre12 = _expand_i_rows(A12, ti, t * ti) + _tile_j(B12, ti) + bf1[...]
    pre21 = _expand_i_rows(A21, ti, t * ti) + _tile_j(B21, ti) + bf1[...]
    u12 = _swish(jnp.dot(_swish(pre12), Wf2[...],
                         preferred_element_type=jnp.float32) + bf2[...])
    u21 = _swish(jnp.dot(_swish(pre21), Wf2[...],
                         preferred_element_type=jnp.float32) + bf2[...])
    nf = u12 + u21
    ns_out[...] = jnp.dot(nf, Ws1_mid[...], preferred_element_type=jnp.float32)
    nk_out[...] = jnp.dot(nf, Wk1_top[...], preferred_element_type=jnp.float32)


def _dotT(lhs, rhs):
    return jax.lax.dot_general(lhs, rhs, (((0,), (0,)), ((), ())),
                               preferred_element_type=jnp.float32)


def _colsum(x):
    return jnp.sum(x, axis=0, keepdims=True)


def _switch(R):
    tt = jnp.clip(R - (_CUTOFF - 1.0), 0.0, 1.0)
    return 1.0 - tt * tt * tt * (tt * (tt * 6.0 - 15.0) + 10.0)


def _stageB_kernel(ns, nk, distl, distr, c1, c2, m1, d1, q1, m2, d2, q2,
                   widths, Ws1_G, Ws1_D, Wk1_D, bs1, bk1, Ws2, bs2, Wk2, bk2,
                   Ws3, bs3, Wk3, bk3, out, *, ti):
    t = pl.program_id(1)
    pb = ti * 96
    Rl = distl[0, 0]
    r_i = jax.lax.broadcasted_iota(jnp.int32, (ti, pb), 1)
    a_i = jax.lax.broadcasted_iota(jnp.int32, (ti, pb), 0)
    Eip = (r_i // 96 == a_i).astype(jnp.float32)
    r_j = jax.lax.broadcasted_iota(jnp.int32, (96, pb), 1)
    j_j = jax.lax.broadcasted_iota(jnp.int32, (96, pb), 0)
    Ejp = (r_j % 96 == j_j).astype(jnp.float32)
    c1l = _dotT(c1[0], Eip)
    m1l = _dotT(m1[0], Eip)
    d1l = _dotT(d1[0], Eip)
    q1l = _dotT(q1[0], Eip)
    c2l = _dotT(c2[0], Ejp)
    m2l = _dotT(m2[0], Ejp)
    d2l = _dotT(d2[0], Ejp)
    q2l = _dotT(q2[0], Ejp)
    Rx = (c2l - c1l) / Rl
    D1 = _colsum(d1l * Rx)
    D2 = _colsum(d2l * Rx)
    dd = _colsum(d1l * d2l)
    qq = _colsum(q1l * q2l)
    Q1v = jnp.concatenate(
        [_colsum(q1l[3 * k:3 * k + 3, :] * Rx) for k in range(3)], axis=0)
    Q2v = jnp.concatenate(
        [_colsum(q2l[3 * k:3 * k + 3, :] * Rx) for k in range(3)], axis=0)
    Q1R2 = _colsum(Q1v * Rx)
    Q2R2 = _colsum(Q2v * Rx)
    Q1d2 = _colsum(Q1v * d2l)
    Q2d1 = _colsum(Q2v * d1l)
    QQv = _colsum(Q1v * Q2v)
    G = jnp.concatenate([
        m1l * m2l,
        D1 * m2l - D2 * m1l,
        dd,
        -(D1 * D2),
        2.0 * Q1d2 - 2.0 * Q2d1,
        Q1R2 * m2l + Q2R2 * m1l,
        2.0 * qq,
        -4.0 * QQv,
        -Q1R2 * D2 + Q2R2 * D1,
        Q1R2 * Q2R2,
    ], axis=0)
    swl = _switch(Rl)
    distf = jnp.exp(-(Rl * Rl) * widths[...]) * swl
    s1 = _swish(ns[...] + _dotT(G, Ws1_G[...]) + _dotT(distf, Ws1_D[...])
                + bs1[...])
    s2 = _swish(jnp.dot(s1, Ws2[...], preferred_element_type=jnp.float32) + bs2[...])
    k1 = _swish(nk[...] + _dotT(distf, Wk1_D[...]) + bk1[...])
    k2 = _swish(jnp.dot(k1, Wk2[...], preferred_element_type=jnp.float32) + bk2[...])
    R = distr[0]
    R2 = R * R
    sw = _switch(R)
    s_out = _softplus(jnp.dot(s2, Ws3[...], preferred_element_type=jnp.float32)
                      + bs3[...]) * sw
    k_out = _softplus(jnp.dot(k2, Wk3[...], preferred_element_type=jnp.float32)
                      + bk3[...])
    S2s = s_out[:, 0:1]
    S2a = s_out[:, 1:2]
    K1s = k_out[:, 0:1]
    K2s = k_out[:, 1:2]
    Ka = k_out[:, 2:3]
    V = K1s * S2s / R + K2s * S2s / R2 - Ka * S2a
    e = jnp.sum(V, axis=(0, 1), keepdims=True)[None]

    @pl.when(t == 0)
    def _init():
        out[...] = e

    @pl.when(t != 0)
    def _acc():
        out[...] = out[...] + e


def kernel(nodes_1, nodes_2, edge_index_1, edge_index_2, coords_1, coords_2,
           distance_matrices, monopoles_1, dipoles_1, quadrupoles_1,
           monopoles_2, dipoles_2, quadrupoles_2, batch_size, params):
    B, n1, n2 = distance_matrices.shape
    TI = 16
    T = n1 // TI
    PB = TI * n2
    f32 = jnp.float32

    p = params
    row = lambda v: v.reshape(1, -1).astype(f32)
    e1s = edge_index_1[0].reshape(-1, 1)
    e1r = edge_index_1[1].reshape(-1, 1)
    e2s = edge_index_2[0].reshape(-1, 1)
    e2r = edge_index_2[1].reshape(-1, 1)

    full = lambda a: pl.BlockSpec(a.shape, lambda *_: (0,) * a.ndim)

    Ws1 = p['Ws1']
    Wk1 = p['Wk1']
    Ws1_G = Ws1[:10, :]
    Ws1_mid = Ws1[10:138, :]
    Ws1_D = Ws1[138:143, :]
    Wk1_top = Wk1[:128, :]
    Wk1_D = Wk1[128:133, :]

    stageA_inputs = (nodes_1, nodes_2, e1s, e1r, e2s, e2r,
                     p['We'], row(p['be']), p['Wu'], row(p['bu']),
                     p['Wf1'], row(p['bf1']), p['Wf2'], row(p['bf2']),
                     Ws1_mid, Wk1_top)
    ns_all, nk_all = pl.pallas_call(
        functools.partial(_stageA_kernel, ti=TI),
        grid=(T,),
        in_specs=[full(a) for a in stageA_inputs],
        out_specs=[
            pl.BlockSpec((PB, 128), lambda t: (t, 0)),
            pl.BlockSpec((PB, 128), lambda t: (t, 0)),
        ],
        out_shape=[
            jax.ShapeDtypeStruct((n1 * n2, 128), f32),
            jax.ShapeDtypeStruct((n1 * n2, 128), f32),
        ],
    )(*stageA_inputs)

    dist3 = distance_matrices.reshape(B, n1 * n2, 1)
    dist4 = distance_matrices.reshape(B, T, 1, PB)
    q1f = quadrupoles_1.reshape(B, n1, 9)
    q2f = quadrupoles_2.reshape(B, n2, 9)

    ib = lambda C: pl.BlockSpec((1, TI, C), lambda b, t: (b, t, 0))
    jb = lambda C: pl.BlockSpec((1, n2, C), lambda b, t: (b, 0, 0))
    stageB_data = (ns_all, nk_all, dist4, dist3, coords_1, coords_2,
                   monopoles_1, dipoles_1, q1f, monopoles_2, dipoles_2, q2f)
    stageB_w = (jnp.asarray(_WIDTHS).reshape(5, 1),
                Ws1_G, Ws1_D, Wk1_D, row(p['bs1']), row(p['bk1']),
                p['Ws2'], row(p['bs2']), p['Wk2'], row(p['bk2']),
                p['Ws3'], row(p['bs3']), p['Wk3'], row(p['bk3']))
    energies = pl.pallas_call(
        functools.partial(_stageB_kernel, ti=TI),
        grid=(B, T),
        in_specs=[
            pl.BlockSpec((PB, 128), lambda b, t: (t, 0)),
            pl.BlockSpec((PB, 128), lambda b, t: (t, 0)),
            pl.BlockSpec((1, 1, 1, PB), lambda b, t: (b, t, 0, 0)),
            pl.BlockSpec((1, PB, 1), lambda b, t: (b, t, 0)),
            ib(3), jb(3), ib(1), ib(3), ib(9), jb(1), jb(3), jb(9),
        ] + [full(a) for a in stageB_w],
        out_specs=pl.BlockSpec((1, 1, 1), lambda b, t: (b, 0, 0)),
        out_shape=jax.ShapeDtypeStruct((B, 1, 1), f32),
    )(*stageB_data, *stageB_w)

    return energies.reshape(B)


# speedup vs baseline: 38.5333x; 1.0620x over previous
import functools

import jax
import jax.numpy as jnp
import numpy as np
from jax.experimental import pallas as pl

_WIDTHS = np.logspace(-1, 0, 5).astype(np.float32)
_CUTOFF = 10.0


def _swish(x):
    return x * (1.0 / (1.0 + jnp.exp(-x)))


def _softplus(x):
    return jnp.maximum(x, 0.0) + jnp.log1p(jnp.exp(-jnp.abs(x)))


def _rowsum(x):
    return jnp.sum(x, axis=-1, keepdims=True)


def _topo(nodes, snd, rcv, We, be, Wu, bu):
    E = snd.shape[0]
    n = nodes.shape[0]
    h = _swish(jnp.dot(nodes, We, preferred_element_type=jnp.float32) + be)
    col = jax.lax.broadcasted_iota(jnp.int32, (E, n), 1)
    oh_s = (col == snd).astype(jnp.float32)
    oh_r = (col == rcv).astype(jnp.float32)
    hs = jnp.dot(oh_s, h, preferred_element_type=jnp.float32)
    agg = jax.lax.dot_general(oh_r, hs, (((0,), (0,)), ((), ())),
                              preferred_element_type=jnp.float32)
    hc = jnp.concatenate([h, agg], axis=1)
    return _swish(jnp.dot(hc, Wu, preferred_element_type=jnp.float32) + bu)


def _expand_i_rows(mat, ti, offset):
    rows = ti * 96
    r = jax.lax.broadcasted_iota(jnp.int32, (rows, mat.shape[0]), 0)
    c = jax.lax.broadcasted_iota(jnp.int32, (rows, mat.shape[0]), 1)
    ei = (offset + r // 96 == c).astype(jnp.float32)
    return jnp.dot(ei, mat, preferred_element_type=jnp.float32)


def _tile_j(mat, reps):
    return jnp.concatenate([mat] * reps, axis=0)


def _stageA_kernel(nodes_1, nodes_2, e1s, e1r, e2s, e2r, We, be, Wu, bu,
                   Wf1, bf1, Wf2, bf2, Ws1_mid, Wk1_top, ns_out, nk_out,
                   *, ti):
    t = pl.program_id(0)
    h1 = _topo(nodes_1[...], e1s[...], e1r[...], We[...], be[...], Wu[...], bu[...])
    h2 = _topo(nodes_2[...], e2s[...], e2r[...], We[...], be[...], Wu[...], bu[...])
    node_size = h1.shape[1]
    Wf1_top = Wf1[:node_size, :]
    Wf1_bot = Wf1[node_size:, :]
    A12 = jnp.dot(h1, Wf1_top, preferred_element_type=jnp.float32)
    A21 = jnp.dot(h1, Wf1_bot, preferred_element_type=jnp.float32)
    B12 = jnp.dot(h2, Wf1_bot, preferred_element_type=jnp.float32)
    B21 = jnp.dot(h2, Wf1_top, preferred_element_type=jnp.float32)
    pre12 = _expand_i_rows(A12, ti, t * ti) + _tile_j(B12, ti) + bf1[...]
    pre21 = _expand_i_rows(A21, ti, t * ti) + _tile_j(B21, ti) + bf1[...]
    u12 = _swish(jnp.dot(_swish(pre12), Wf2[...],
                         preferred_element_type=jnp.float32) + bf2[...])
    u21 = _swish(jnp.dot(_swish(pre21), Wf2[...],
                         preferred_element_type=jnp.float32) + bf2[...])
    nf = u12 + u21
    ns_out[...] = jnp.dot(nf, Ws1_mid[...], preferred_element_type=jnp.float32)
    nk_out[...] = jnp.dot(nf, Wk1_top[...], preferred_element_type=jnp.float32)


def _dotT(lhs, rhs):
    return jax.lax.dot_general(lhs, rhs, (((0,), (0,)), ((), ())),
                               preferred_element_type=jnp.float32)


def _colsum(x):
    return jnp.sum(x, axis=0, keepdims=True)


def _switch(R):
    tt = jnp.clip(R - (_CUTOFF - 1.0), 0.0, 1.0)
    return 1.0 - tt * tt * tt * (tt * (tt * 6.0 - 15.0) + 10.0)


def _stageB_kernel(ns, nk, distl, distr, c1, c2, m1, d1, q1, m2, d2, q2,
                   widths, Ws1_G, Ws1_D, Wk1_D, bs1, bk1, Ws2, bs2, Wk2, bk2,
                   Ws3, bs3, Wk3, bk3, out, *, ti):
    pb = ti * 96
    Rl = distl[0, 0]
    r_i = jax.lax.broadcasted_iota(jnp.int32, (ti, pb), 1)
    a_i = jax.lax.broadcasted_iota(jnp.int32, (ti, pb), 0)
    Eip = (r_i // 96 == a_i).astype(jnp.float32)
    r_j = jax.lax.broadcasted_iota(jnp.int32, (96, pb), 1)
    j_j = jax.lax.broadcasted_iota(jnp.int32, (96, pb), 0)
    Ejp = (r_j % 96 == j_j).astype(jnp.float32)
    c1l = _dotT(c1[0], Eip)
    m1l = _dotT(m1[0], Eip)
    d1l = _dotT(d1[0], Eip)
    q1l = _dotT(q1[0], Eip)
    c2l = _dotT(c2[0], Ejp)
    m2l = _dotT(m2[0], Ejp)
    d2l = _dotT(d2[0], Ejp)
    q2l = _dotT(q2[0], Ejp)
    Rx = (c2l - c1l) / Rl
    D1 = _colsum(d1l * Rx)
    D2 = _colsum(d2l * Rx)
    dd = _colsum(d1l * d2l)
    qq = _colsum(q1l * q2l)
    Q1v = jnp.concatenate(
        [_colsum(q1l[3 * k:3 * k + 3, :] * Rx) for k in range(3)], axis=0)
    Q2v = jnp.concatenate(
        [_colsum(q2l[3 * k:3 * k + 3, :] * Rx) for k in range(3)], axis=0)
    Q1R2 = _colsum(Q1v * Rx)
    Q2R2 = _colsum(Q2v * Rx)
    Q1d2 = _colsum(Q1v * d2l)
    Q2d1 = _colsum(Q2v * d1l)
    QQv = _colsum(Q1v * Q2v)
    G = jnp.concatenate([
        m1l * m2l,
        D1 * m2l - D2 * m1l,
        dd,
        -(D1 * D2),
        2.0 * Q1d2 - 2.0 * Q2d1,
        Q1R2 * m2l + Q2R2 * m1l,
        2.0 * qq,
        -4.0 * QQv,
        -Q1R2 * D2 + Q2R2 * D1,
        Q1R2 * Q2R2,
    ], axis=0)
    swl = _switch(Rl)
    distf = jnp.exp(-(Rl * Rl) * widths[...]) * swl
    s1 = _swish(ns[...] + _dotT(G, Ws1_G[...]) + _dotT(distf, Ws1_D[...])
                + bs1[...])
    s2 = _swish(jnp.dot(s1, Ws2[...], preferred_element_type=jnp.float32) + bs2[...])
    k1 = _swish(nk[...] + _dotT(distf, Wk1_D[...]) + bk1[...])
    k2 = _swish(jnp.dot(k1, Wk2[...], preferred_element_type=jnp.float32) + bk2[...])
    R = distr[0]
    R2 = R * R
    sw = _switch(R)
    s_out = _softplus(jnp.dot(s2, Ws3[...], preferred_element_type=jnp.float32)
                      + bs3[...]) * sw
    k_out = _softplus(jnp.dot(k2, Wk3[...], preferred_element_type=jnp.float32)
                      + bk3[...])
    S2s = s_out[:, 0:1]
    S2a = s_out[:, 1:2]
    K1s = k_out[:, 0:1]
    K2s = k_out[:, 1:2]
    Ka = k_out[:, 2:3]
    V = K1s * S2s / R + K2s * S2s / R2 - Ka * S2a
    out[...] = jnp.sum(V, axis=(0, 1), keepdims=True)[None, None]


def kernel(nodes_1, nodes_2, edge_index_1, edge_index_2, coords_1, coords_2,
           distance_matrices, monopoles_1, dipoles_1, quadrupoles_1,
           monopoles_2, dipoles_2, quadrupoles_2, batch_size, params):
    B, n1, n2 = distance_matrices.shape
    TI = 32
    T = n1 // TI
    PB = TI * n2
    f32 = jnp.float32

    p = params
    row = lambda v: v.reshape(1, -1).astype(f32)
    e1s = edge_index_1[0].reshape(-1, 1)
    e1r = edge_index_1[1].reshape(-1, 1)
    e2s = edge_index_2[0].reshape(-1, 1)
    e2r = edge_index_2[1].reshape(-1, 1)

    full = lambda a: pl.BlockSpec(a.shape, lambda *_: (0,) * a.ndim)

    Ws1 = p['Ws1']
    Wk1 = p['Wk1']
    Ws1_G = Ws1[:10, :]
    Ws1_mid = Ws1[10:138, :]
    Ws1_D = Ws1[138:143, :]
    Wk1_top = Wk1[:128, :]
    Wk1_D = Wk1[128:133, :]

    stageA_inputs = (nodes_1, nodes_2, e1s, e1r, e2s, e2r,
                     p['We'], row(p['be']), p['Wu'], row(p['bu']),
                     p['Wf1'], row(p['bf1']), p['Wf2'], row(p['bf2']),
                     Ws1_mid, Wk1_top)
    ns_all, nk_all = pl.pallas_call(
        functools.partial(_stageA_kernel, ti=TI),
        grid=(T,),
        in_specs=[full(a) for a in stageA_inputs],
        out_specs=[
            pl.BlockSpec((PB, 128), lambda t: (t, 0)),
            pl.BlockSpec((PB, 128), lambda t: (t, 0)),
        ],
        out_shape=[
            jax.ShapeDtypeStruct((n1 * n2, 128), f32),
            jax.ShapeDtypeStruct((n1 * n2, 128), f32),
        ],
    )(*stageA_inputs)

    dist3 = distance_matrices.reshape(B, n1 * n2, 1)
    dist4 = distance_matrices.reshape(B, T, 1, PB)
    q1f = quadrupoles_1.reshape(B, n1, 9)
    q2f = quadrupoles_2.reshape(B, n2, 9)

    ib = lambda C: pl.BlockSpec((1, TI, C), lambda t, b: (b, t, 0))
    jb = lambda C: pl.BlockSpec((1, n2, C), lambda t, b: (b, 0, 0))
    stageB_data = (ns_all, nk_all, dist4, dist3, coords_1, coords_2,
                   monopoles_1, dipoles_1, q1f, monopoles_2, dipoles_2, q2f)
    stageB_w = (jnp.asarray(_WIDTHS).reshape(5, 1),
                Ws1_G, Ws1_D, Wk1_D, row(p['bs1']), row(p['bk1']),
                p['Ws2'], row(p['bs2']), p['Wk2'], row(p['bk2']),
                p['Ws3'], row(p['bs3']), p['Wk3'], row(p['bk3']))
    partials = pl.pallas_call(
        functools.partial(_stageB_kernel, ti=TI),
        grid=(T, B),
        in_specs=[
            pl.BlockSpec((PB, 128), lambda t, b: (t, 0)),
            pl.BlockSpec((PB, 128), lambda t, b: (t, 0)),
            pl.BlockSpec((1, 1, 1, PB), lambda t, b: (b, t, 0, 0)),
            pl.BlockSpec((1, PB, 1), lambda t, b: (b, t, 0)),
            ib(3), jb(3), ib(1), ib(3), ib(9), jb(1), jb(3), jb(9),
        ] + [full(a) for a in stageB_w],
        out_specs=pl.BlockSpec((1, 1, 1, 1), lambda t, b: (b, t, 0, 0)),
        out_shape=jax.ShapeDtypeStruct((B, T, 1, 1), f32),
    )(*stageB_data, *stageB_w)

    return partials.reshape(B, T).sum(axis=1)
